# baseline (device time: 115753 ns/iter reference)
import jax
import jax.numpy as jnp
from jax import lax
from jax.experimental import pallas as pl
from jax.experimental.pallas import tpu as pltpu

N_DEV = 8
B_PER = 2
SQ = 128
D = 512
H_PER = 8
DH = 64
ROWS = B_PER * SQ


def kernel(x, Wq, Wo, Wk, Wv):
    def body(x_ref, wq_ref, wo_ref, wk_ref, wv_ref, out_ref,
             xg_ref, rs_send_ref, rs_recv_ref,
             ag_send_sems, ag_recv_sems, rs_send_sems, rs_recv_sems):
        my = lax.axis_index("i")
        left = lax.rem(my + N_DEV - 1, N_DEV)
        right = lax.rem(my + 1, N_DEV)

        barrier_sem = pltpu.get_barrier_semaphore()
        for nbr in (left, right):
            pl.semaphore_signal(
                barrier_sem, inc=1,
                device_id=(nbr,), device_id_type=pl.DeviceIdType.MESH,
            )
        pl.semaphore_wait(barrier_sem, 2)

        wq = wq_ref[...].astype(jnp.bfloat16)
        wk = wk_ref[...].astype(jnp.bfloat16)
        wv = wv_ref[...].astype(jnp.bfloat16)
        wo = wo_ref[...].astype(jnp.bfloat16)

        own = x_ref[...].reshape(ROWS, D).astype(jnp.bfloat16)
        xg_ref[pl.ds(my, 1)] = own[None]

        for t in range(N_DEV - 1):
            j = lax.rem(my - t + N_DEV, N_DEV)
            rdma = pltpu.make_async_remote_copy(
                src_ref=xg_ref.at[j],
                dst_ref=xg_ref.at[j],
                send_sem=ag_send_sems.at[t],
                recv_sem=ag_recv_sems.at[t],
                device_id=(right,),
                device_id_type=pl.DeviceIdType.MESH,
            )
            rdma.start()
            rdma.wait()

        def contrib(j):
            xj = xg_ref[pl.ds(j, 1)].reshape(ROWS, D)
            q = lax.dot(xj, wq, preferred_element_type=jnp.float32)
            k = lax.dot(xj, wk, preferred_element_type=jnp.float32)
            v = lax.dot(xj, wv, preferred_element_type=jnp.float32)
            qh = q.reshape(B_PER, SQ, H_PER, DH).transpose(0, 2, 1, 3) \
                  .reshape(B_PER * H_PER, SQ, DH).astype(jnp.bfloat16)
            kh = k.reshape(B_PER, SQ, H_PER, DH).transpose(0, 2, 1, 3) \
                  .reshape(B_PER * H_PER, SQ, DH).astype(jnp.bfloat16)
            vh = v.reshape(B_PER, SQ, H_PER, DH).transpose(0, 2, 1, 3) \
                  .reshape(B_PER * H_PER, SQ, DH).astype(jnp.bfloat16)
            s = lax.dot_general(
                qh, kh, (((2,), (2,)), ((0,), (0,))),
                preferred_element_type=jnp.float32,
            ) * 0.125
            m = jnp.max(s, axis=-1, keepdims=True)
            p = jnp.exp(s - m)
            l = jnp.sum(p, axis=-1, keepdims=True)
            pb = (p / l).astype(jnp.bfloat16)
            o = lax.dot_general(
                pb, vh, (((2,), (1,)), ((0,), (0,))),
                preferred_element_type=jnp.float32,
            )
            ob = o.reshape(B_PER, H_PER, SQ, DH).transpose(0, 2, 1, 3) \
                  .reshape(ROWS, D).astype(jnp.bfloat16)
            return lax.dot(ob, wo, preferred_element_type=jnp.float32)

        for t in range(N_DEV - 1):
            j = lax.rem(my - 1 - t + N_DEV, N_DEV)
            acc = contrib(j)
            if t > 0:
                acc = acc + rs_recv_ref[t - 1]
            rs_send_ref[...] = acc
            rdma = pltpu.make_async_remote_copy(
                src_ref=rs_send_ref,
                dst_ref=rs_recv_ref.at[t],
                send_sem=rs_send_sems.at[t],
                recv_sem=rs_recv_sems.at[t],
                device_id=(right,),
                device_id_type=pl.DeviceIdType.MESH,
            )
            rdma.start()
            rdma.wait()

        total = contrib(my) + rs_recv_ref[N_DEV - 2]
        out_ref[...] = total.reshape(B_PER, SQ, D)

    return pl.pallas_call(
        body,
        out_shape=jax.ShapeDtypeStruct((B_PER, SQ, D), jnp.float32),
        in_specs=[pl.BlockSpec(memory_space=pltpu.VMEM)] * 5,
        out_specs=pl.BlockSpec(memory_space=pltpu.VMEM),
        scratch_shapes=[
            pltpu.VMEM((N_DEV, ROWS, D), jnp.bfloat16),
            pltpu.VMEM((ROWS, D), jnp.float32),
            pltpu.VMEM((N_DEV - 1, ROWS, D), jnp.float32),
            pltpu.SemaphoreType.DMA((N_DEV - 1,)),
            pltpu.SemaphoreType.DMA((N_DEV - 1,)),
            pltpu.SemaphoreType.DMA((N_DEV - 1,)),
            pltpu.SemaphoreType.DMA((N_DEV - 1,)),
        ],
        compiler_params=pltpu.CompilerParams(collective_id=0),
    )(x, Wq, Wo, Wk, Wv)


# device time: 56839 ns/iter; 2.0365x vs baseline; 2.0365x over previous
import jax
import jax.numpy as jnp
from jax import lax
from jax.experimental import pallas as pl
from jax.experimental.pallas import tpu as pltpu

N_DEV = 8
B_PER = 2
SQ = 128
D = 512
H_PER = 8
DH = 64
ROWS = B_PER * SQ


def kernel(x, Wq, Wo, Wk, Wv):
    def body(x_ref, wq_ref, wo_ref, wk_ref, wv_ref, out_ref,
             xg_ref, rs_send_ref, rs_recv_ref,
             ag_send_sems, ag_recv_sems, rs_send_sems, rs_recv_sems):
        my = lax.axis_index("i")
        left = lax.rem(my + N_DEV - 1, N_DEV)
        right = lax.rem(my + 1, N_DEV)

        barrier_sem = pltpu.get_barrier_semaphore()
        for nbr in (left, right):
            pl.semaphore_signal(
                barrier_sem, inc=1,
                device_id=(nbr,), device_id_type=pl.DeviceIdType.MESH,
            )
        pl.semaphore_wait(barrier_sem, 2)

        wq = wq_ref[...].astype(jnp.bfloat16)
        wk = wk_ref[...].astype(jnp.bfloat16)
        wv = wv_ref[...].astype(jnp.bfloat16)
        wo = wo_ref[...].astype(jnp.bfloat16)

        own = x_ref[...].reshape(ROWS, D).astype(jnp.bfloat16)
        xg_ref[pl.ds(my, 1)] = own[None]

        def contrib(j):
            xj = xg_ref[pl.ds(j, 1)].reshape(ROWS, D)
            q = lax.dot(xj, wq, preferred_element_type=jnp.float32)
            k = lax.dot(xj, wk, preferred_element_type=jnp.float32)
            v = lax.dot(xj, wv, preferred_element_type=jnp.float32)
            qh = q.reshape(B_PER, SQ, H_PER, DH).transpose(0, 2, 1, 3) \
                  .reshape(B_PER * H_PER, SQ, DH).astype(jnp.bfloat16)
            kh = k.reshape(B_PER, SQ, H_PER, DH).transpose(0, 2, 1, 3) \
                  .reshape(B_PER * H_PER, SQ, DH).astype(jnp.bfloat16)
            vh = v.reshape(B_PER, SQ, H_PER, DH).transpose(0, 2, 1, 3) \
                  .reshape(B_PER * H_PER, SQ, DH).astype(jnp.bfloat16)
            s = lax.dot_general(
                qh, kh, (((2,), (2,)), ((0,), (0,))),
                preferred_element_type=jnp.float32,
            ) * 0.125
            m = jnp.max(s, axis=-1, keepdims=True)
            p = jnp.exp(s - m)
            l = jnp.sum(p, axis=-1, keepdims=True)
            pb = (p / l).astype(jnp.bfloat16)
            o = lax.dot_general(
                pb, vh, (((2,), (1,)), ((0,), (0,))),
                preferred_element_type=jnp.float32,
            )
            ob = o.reshape(B_PER, H_PER, SQ, DH).transpose(0, 2, 1, 3) \
                  .reshape(ROWS, D).astype(jnp.bfloat16)
            return lax.dot(ob, wo, preferred_element_type=jnp.float32)

        def ag_rdma(t):
            j = lax.rem(my - t + N_DEV, N_DEV)
            return pltpu.make_async_remote_copy(
                src_ref=xg_ref.at[j],
                dst_ref=xg_ref.at[j],
                send_sem=ag_send_sems.at[t],
                recv_sem=ag_recv_sems.at[t],
                device_id=(right,),
                device_id_type=pl.DeviceIdType.MESH,
            )

        def rs_rdma(t):
            return pltpu.make_async_remote_copy(
                src_ref=rs_send_ref.at[t],
                dst_ref=rs_recv_ref.at[t],
                send_sem=rs_send_sems.at[t],
                recv_sem=rs_recv_sems.at[t],
                device_id=(right,),
                device_id_type=pl.DeviceIdType.MESH,
            )

        ags = [ag_rdma(t) for t in range(N_DEV - 1)]
        rss = [rs_rdma(t) for t in range(N_DEV - 1)]

        ags[0].start()
        own_acc = contrib(my)

        for t in range(N_DEV - 1):
            j = lax.rem(my - 1 - t + N_DEV, N_DEV)
            ags[t].wait_recv()
            if t + 1 < N_DEV - 1:
                ags[t + 1].start()
            acc = contrib(j)
            if t > 0:
                rss[t - 1].wait_recv()
                acc = acc + rs_recv_ref[t - 1].astype(jnp.float32)
            rs_send_ref[pl.ds(t, 1)] = acc.astype(jnp.bfloat16)[None]
            rss[t].start()

        rss[N_DEV - 2].wait_recv()
        total = own_acc + rs_recv_ref[N_DEV - 2].astype(jnp.float32)
        out_ref[...] = total.reshape(B_PER, SQ, D)

        for t in range(N_DEV - 1):
            ags[t].wait_send()
            rss[t].wait_send()

    return pl.pallas_call(
        body,
        out_shape=jax.ShapeDtypeStruct((B_PER, SQ, D), jnp.float32),
        in_specs=[pl.BlockSpec(memory_space=pltpu.VMEM)] * 5,
        out_specs=pl.BlockSpec(memory_space=pltpu.VMEM),
        scratch_shapes=[
            pltpu.VMEM((N_DEV, ROWS, D), jnp.bfloat16),
            pltpu.VMEM((N_DEV - 1, ROWS, D), jnp.bfloat16),
            pltpu.VMEM((N_DEV - 1, ROWS, D), jnp.bfloat16),
            pltpu.SemaphoreType.DMA((N_DEV - 1,)),
            pltpu.SemaphoreType.DMA((N_DEV - 1,)),
            pltpu.SemaphoreType.DMA((N_DEV - 1,)),
            pltpu.SemaphoreType.DMA((N_DEV - 1,)),
        ],
        compiler_params=pltpu.CompilerParams(collective_id=0),
    )(x, Wq, Wo, Wk, Wv)


# device time: 42989 ns/iter; 2.6926x vs baseline; 1.3222x over previous
import jax
import jax.numpy as jnp
from jax import lax
from jax.experimental import pallas as pl
from jax.experimental.pallas import tpu as pltpu

N_DEV = 8
B_PER = 2
SQ = 128
D = 512
H_PER = 8
DH = 64
ROWS = B_PER * SQ


def kernel(x, Wq, Wo, Wk, Wv):
    def body(x_ref, wq_ref, wo_ref, wk_ref, wv_ref, out_ref,
             xg_ref, acc_send_ref, acc_recv_ref,
             x_send_sems, x_recv_sems, a_send_sems, a_recv_sems):
        my = lax.axis_index("i")

        barrier_sem = pltpu.get_barrier_semaphore()
        for k in range(1, N_DEV):
            p = lax.rem(my + k, N_DEV)
            pl.semaphore_signal(
                barrier_sem, inc=1,
                device_id=(p,), device_id_type=pl.DeviceIdType.MESH,
            )
        pl.semaphore_wait(barrier_sem, N_DEV - 1)

        own = x_ref[...].reshape(ROWS, D).astype(jnp.bfloat16)
        xg_ref[pl.ds(my, 1)] = own[None]

        x_sends = []
        for k in range(1, N_DEV):
            p = lax.rem(my + k, N_DEV)
            s = pltpu.make_async_remote_copy(
                src_ref=xg_ref.at[my],
                dst_ref=xg_ref.at[my],
                send_sem=x_send_sems.at[k],
                recv_sem=x_recv_sems.at[my],
                device_id=(p,),
                device_id_type=pl.DeviceIdType.MESH,
            )
            s.start()
            x_sends.append(s)

        wq = (wq_ref[...] * 0.125).astype(jnp.bfloat16)
        wk = wk_ref[...].astype(jnp.bfloat16)
        wv = wv_ref[...].astype(jnp.bfloat16)
        wo = wo_ref[...].astype(jnp.bfloat16)

        def contrib(j):
            xj = xg_ref[pl.ds(j, 1)].reshape(ROWS, D)
            q = lax.dot(xj, wq, preferred_element_type=jnp.float32)
            k_ = lax.dot(xj, wk, preferred_element_type=jnp.float32)
            v_ = lax.dot(xj, wv, preferred_element_type=jnp.float32)
            qh = q.reshape(B_PER, SQ, H_PER, DH).transpose(0, 2, 1, 3) \
                  .reshape(B_PER * H_PER, SQ, DH).astype(jnp.bfloat16)
            kh = k_.reshape(B_PER, SQ, H_PER, DH).transpose(0, 2, 1, 3) \
                   .reshape(B_PER * H_PER, SQ, DH).astype(jnp.bfloat16)
            vh = v_.reshape(B_PER, SQ, H_PER, DH).transpose(0, 2, 1, 3) \
                   .reshape(B_PER * H_PER, SQ, DH).astype(jnp.bfloat16)
            s = lax.dot_general(
                qh, kh, (((2,), (2,)), ((0,), (0,))),
                preferred_element_type=jnp.float32,
            )
            m = jnp.max(s, axis=-1, keepdims=True)
            p = jnp.exp(s - m)
            l = jnp.sum(p, axis=-1, keepdims=True)
            pb = (p / l).astype(jnp.bfloat16)
            o = lax.dot_general(
                pb, vh, (((2,), (1,)), ((0,), (0,))),
                preferred_element_type=jnp.float32,
            )
            ob = o.reshape(B_PER, H_PER, SQ, DH).transpose(0, 2, 1, 3) \
                  .reshape(ROWS, D).astype(jnp.bfloat16)
            return lax.dot(ob, wo, preferred_element_type=jnp.float32)

        own_acc = contrib(my)

        a_sends = []
        for k in range(1, N_DEV):
            j = lax.rem(my - k + N_DEV, N_DEV)
            recv = pltpu.make_async_remote_copy(
                src_ref=xg_ref.at[j],
                dst_ref=xg_ref.at[j],
                send_sem=x_send_sems.at[k],
                recv_sem=x_recv_sems.at[j],
                device_id=(j,),
                device_id_type=pl.DeviceIdType.MESH,
            )
            recv.wait_recv()
            acc_send_ref[pl.ds(j, 1)] = contrib(j).astype(jnp.bfloat16)[None]
            s = pltpu.make_async_remote_copy(
                src_ref=acc_send_ref.at[j],
                dst_ref=acc_recv_ref.at[my],
                send_sem=a_send_sems.at[k],
                recv_sem=a_recv_sems.at[my],
                device_id=(j,),
                device_id_type=pl.DeviceIdType.MESH,
            )
            s.start()
            a_sends.append(s)

        total = own_acc
        for k in range(1, N_DEV):
            j = lax.rem(my + k, N_DEV)
            recv = pltpu.make_async_remote_copy(
                src_ref=acc_send_ref.at[j],
                dst_ref=acc_recv_ref.at[j],
                send_sem=a_send_sems.at[k],
                recv_sem=a_recv_sems.at[j],
                device_id=(j,),
                device_id_type=pl.DeviceIdType.MESH,
            )
            recv.wait_recv()
            total = total + acc_recv_ref[j].astype(jnp.float32)

        out_ref[...] = total.reshape(B_PER, SQ, D)

        for s in x_sends:
            s.wait_send()
        for s in a_sends:
            s.wait_send()

    return pl.pallas_call(
        body,
        out_shape=jax.ShapeDtypeStruct((B_PER, SQ, D), jnp.float32),
        in_specs=[pl.BlockSpec(memory_space=pltpu.VMEM)] * 5,
        out_specs=pl.BlockSpec(memory_space=pltpu.VMEM),
        scratch_shapes=[
            pltpu.VMEM((N_DEV, ROWS, D), jnp.bfloat16),
            pltpu.VMEM((N_DEV, ROWS, D), jnp.bfloat16),
            pltpu.VMEM((N_DEV, ROWS, D), jnp.bfloat16),
            pltpu.SemaphoreType.DMA((N_DEV,)),
            pltpu.SemaphoreType.DMA((N_DEV,)),
            pltpu.SemaphoreType.DMA((N_DEV,)),
            pltpu.SemaphoreType.DMA((N_DEV,)),
        ],
        compiler_params=pltpu.CompilerParams(collective_id=0),
    )(x, Wq, Wo, Wk, Wv)
